# baseline (speedup 1.0000x reference)
import functools

import jax
import jax.numpy as jnp
from jax import lax
from jax.experimental import pallas as pl
from jax.experimental.pallas import tpu as pltpu
from jax.experimental.pallas import tpu_sc as plsc

NC = 2
NS = 16
NW = NC * NS

CTOK = 256
PAD_D = 128
TBLK = 8192


def _tc_transpose_body(wt_ref, out_ref):
    out_ref[:, 0:64] = wt_ref[...].T


def _tc_transpose(wt):
    d, v = wt.shape
    grid = (v + TBLK - 1) // TBLK
    return pl.pallas_call(
        _tc_transpose_body,
        out_shape=jax.ShapeDtypeStruct((v, PAD_D), jnp.float32),
        grid=(grid,),
        in_specs=[pl.BlockSpec((d, TBLK), lambda i: (0, i))],
        out_specs=pl.BlockSpec((TBLK, PAD_D), lambda i: (i, 0)),
    )(wt)


def _gather_body(
    idx_hbm, pad_hbm, out_hbm, idx_v, rows_v,
    sem0, sem1, isem0, isem1, ssem0, ssem1,
):
    n_idx_rows = idx_hbm.shape[0]
    wid = lax.axis_index("s") * NC + lax.axis_index("c")
    rows_per_w = n_idx_rows // NW
    iters = rows_per_w // 2
    r_lo = wid * rows_per_w
    sems = (sem0, sem1)
    isems = (isem0, isem1)
    ssems = (ssem0, ssem1)

    def issue_idx(g, b):
        pltpu.async_copy(
            idx_hbm.at[pl.ds(r_lo + g * 2, 2)], idx_v.at[b], isems[b]
        )

    def wait_idx(g, b):
        pltpu.make_async_copy(
            idx_hbm.at[pl.ds(r_lo + g * 2, 2)], idx_v.at[b], isems[b]
        ).wait()

    def fire(g, b):
        for i in range(2):
            pltpu.async_copy(
                pad_hbm.at[idx_v.at[b].at[i]],
                rows_v.at[b].at[pl.ds(i * 128, 128)],
                sems[b],
            )

    def drain(b):
        for i in range(2):
            pltpu.make_async_copy(
                pad_hbm.at[idx_v.at[b].at[i]],
                rows_v.at[b].at[pl.ds(i * 128, 128)],
                sems[b],
            ).wait()

    def store_desc(g, b):
        return pltpu.make_async_copy(
            rows_v.at[b],
            out_hbm.at[pl.ds((r_lo + g * 2) * 128, CTOK)],
            ssems[b],
        )

    issue_idx(0, 0)
    wait_idx(0, 0)
    fire(0, 0)
    issue_idx(1, 1)

    def step(g2, carry):
        for b in range(2):
            g = g2 * 2 + b
            nxt = 1 - b

            @pl.when(g + 1 < iters)
            def _():
                wait_idx(g + 1, nxt)
                @pl.when(g >= 1)
                def _():
                    store_desc(g - 1, nxt).wait()

                fire(g + 1, nxt)

            drain(b)
            @pl.when(g + 2 < iters)
            def _():
                issue_idx(g + 2, b)

            pltpu.async_copy(
                rows_v.at[b],
                out_hbm.at[pl.ds((r_lo + g * 2) * 128, CTOK)],
                ssems[b],
            )
        return carry

    lax.fori_loop(0, iters // 2, step, 0)
    store_desc(iters - 2, 0).wait()
    store_desc(iters - 1, 1).wait()


def _gather_call(idx2d, pad_table):
    n_tok = idx2d.shape[0] * idx2d.shape[1]
    mesh = plsc.VectorSubcoreMesh(
        core_axis_name="c", subcore_axis_name="s", num_cores=NC, num_subcores=NS
    )
    return pl.kernel(
        _gather_body,
        out_type=jax.ShapeDtypeStruct((n_tok, PAD_D), jnp.float32),
        mesh=mesh,
        scratch_types=[
            pltpu.VMEM((2, 2, 128), jnp.int32),
            pltpu.VMEM((2, CTOK, PAD_D), jnp.float32),
            pltpu.SemaphoreType.DMA,
            pltpu.SemaphoreType.DMA,
            pltpu.SemaphoreType.DMA,
            pltpu.SemaphoreType.DMA,
            pltpu.SemaphoreType.DMA,
            pltpu.SemaphoreType.DMA,
        ],
        compiler_params=pltpu.CompilerParams(
            use_tc_tiling_on_sc=True, needs_layout_passes=False
        ),
    )(idx2d, pad_table)


def kernel(token_ids, weight):
    b, l = token_ids.shape
    wt = weight.T
    pad_table = _tc_transpose(wt)
    idx2d = token_ids.astype(jnp.int32).reshape(b * l // 128, 128)
    o128 = _gather_call(idx2d, pad_table)
    return o128[:, :64].reshape(b, l, 64)


# speedup vs baseline: 1.0192x; 1.0192x over previous
import functools

import jax
import jax.numpy as jnp
from jax import lax
from jax.experimental import pallas as pl
from jax.experimental.pallas import tpu as pltpu
from jax.experimental.pallas import tpu_sc as plsc

NC = 2
NS = 16
NW = NC * NS

CTOK = 256
PAD_D = 128
TBLK = 12288


def _tc_transpose_body(wt_ref, out_ref):
    out_ref[:, 0:64] = wt_ref[...].T


def _tc_transpose(wt):
    d, v = wt.shape
    grid = (v + TBLK - 1) // TBLK
    return pl.pallas_call(
        _tc_transpose_body,
        out_shape=jax.ShapeDtypeStruct((v, PAD_D), jnp.float32),
        grid=(grid,),
        in_specs=[pl.BlockSpec((d, TBLK), lambda i: (0, i))],
        out_specs=pl.BlockSpec((TBLK, PAD_D), lambda i: (i, 0)),
    )(wt)


def _gather_body(
    idx_hbm, pad_hbm, out_hbm, idx_v, rows_v,
    sem0, sem1, isem0, isem1, ssem0, ssem1,
):
    n_idx_rows = idx_hbm.shape[0]
    wid = lax.axis_index("s") * NC + lax.axis_index("c")
    rows_per_w = n_idx_rows // NW
    iters = rows_per_w // 2
    r_lo = wid * rows_per_w
    sems = (sem0, sem1)
    isems = (isem0, isem1)
    ssems = (ssem0, ssem1)

    def issue_idx(g, b):
        pltpu.async_copy(
            idx_hbm.at[pl.ds(r_lo + g * 2, 2)], idx_v.at[b], isems[b]
        )

    def wait_idx(g, b):
        pltpu.make_async_copy(
            idx_hbm.at[pl.ds(r_lo + g * 2, 2)], idx_v.at[b], isems[b]
        ).wait()

    def fire(g, b):
        for i in range(2):
            pltpu.async_copy(
                pad_hbm.at[idx_v.at[b].at[i]],
                rows_v.at[b].at[pl.ds(i * 128, 128)],
                sems[b],
            )

    def drain(b):
        for i in range(2):
            pltpu.make_async_copy(
                pad_hbm.at[idx_v.at[b].at[i]],
                rows_v.at[b].at[pl.ds(i * 128, 128)],
                sems[b],
            ).wait()

    def store_desc(g, b):
        return pltpu.make_async_copy(
            rows_v.at[b],
            out_hbm.at[pl.ds((r_lo + g * 2) * 128, CTOK)],
            ssems[b],
        )

    issue_idx(0, 0)
    wait_idx(0, 0)
    fire(0, 0)
    issue_idx(1, 1)

    def step(g2, carry):
        for b in range(2):
            g = g2 * 2 + b
            nxt = 1 - b

            @pl.when(g + 1 < iters)
            def _():
                wait_idx(g + 1, nxt)
                @pl.when(g >= 1)
                def _():
                    store_desc(g - 1, nxt).wait()

                fire(g + 1, nxt)

            drain(b)
            @pl.when(g + 2 < iters)
            def _():
                issue_idx(g + 2, b)

            pltpu.async_copy(
                rows_v.at[b],
                out_hbm.at[pl.ds((r_lo + g * 2) * 128, CTOK)],
                ssems[b],
            )
        return carry

    lax.fori_loop(0, iters // 2, step, 0)
    store_desc(iters - 2, 0).wait()
    store_desc(iters - 1, 1).wait()


def _gather_call(idx2d, pad_table):
    n_tok = idx2d.shape[0] * idx2d.shape[1]
    mesh = plsc.VectorSubcoreMesh(
        core_axis_name="c", subcore_axis_name="s", num_cores=NC, num_subcores=NS
    )
    return pl.kernel(
        _gather_body,
        out_type=jax.ShapeDtypeStruct((n_tok, PAD_D), jnp.float32),
        mesh=mesh,
        scratch_types=[
            pltpu.VMEM((2, 2, 128), jnp.int32),
            pltpu.VMEM((2, CTOK, PAD_D), jnp.float32),
            pltpu.SemaphoreType.DMA,
            pltpu.SemaphoreType.DMA,
            pltpu.SemaphoreType.DMA,
            pltpu.SemaphoreType.DMA,
            pltpu.SemaphoreType.DMA,
            pltpu.SemaphoreType.DMA,
        ],
        compiler_params=pltpu.CompilerParams(
            use_tc_tiling_on_sc=True, needs_layout_passes=False
        ),
    )(idx2d, pad_table)


def kernel(token_ids, weight):
    b, l = token_ids.shape
    wt = weight.T
    pad_table = _tc_transpose(wt)
    idx2d = token_ids.astype(jnp.int32).reshape(b * l // 128, 128)
    o128 = _gather_call(idx2d, pad_table)
    return o128[:, :64].reshape(b, l, 64)
